# Initial kernel scaffold; baseline (speedup 1.0000x reference)
import jax
import jax.numpy as jnp
from jax.experimental import pallas as pl


def kernel(coords, torsions):
    raise NotImplementedError("write your pallas kernel here")


# speedup vs baseline: 10.4214x; 10.4214x over previous
import functools

import jax
import jax.numpy as jnp
from jax import lax
from jax.experimental import pallas as pl
from jax.experimental.pallas import tpu as pltpu
from jax.experimental.pallas import tpu_sc as plsc

_NC = 2
_NS = 16
_NW = _NC * _NS
_L = 16

_CHUNK = 2048


def _rsqrt(x):
    bits = lax.bitcast_convert_type(x, jnp.int32)
    y = lax.bitcast_convert_type(
        jnp.int32(0x5F3759DF) - lax.shift_right_logical(bits, 1), jnp.float32
    )
    for _ in range(3):
        t = x * y
        y = y * (1.5 - 0.5 * t * y)
    return y


def _acos(x):
    ax = jnp.abs(x)
    omx = 1.0 - ax
    s = omx * _rsqrt(omx)
    p = ((-0.0187293 * ax + 0.0742610) * ax - 0.2121144) * ax + 1.5707288
    pos = s * p
    return jnp.where(x >= 0.0, pos, 3.14159265358979 - pos)


def _body(coords_hbm, tors_hbm, out_hbm,
          tors_v, idx_a, idx_b, idx_c, idx_d,
          rows_a, rows_b, rows_c, rows_d, out_v, sem,
          *, n_tors):
    cid_c = lax.axis_index("c")
    cid_s = lax.axis_index("s")
    w = cid_s * _NC + cid_c

    iota = lax.iota(jnp.int32, _L)
    col = [jnp.full((_L,), c, jnp.int32) for c in range(3)]
    idx_bufs = (idx_a, idx_b, idx_c, idx_d)
    row_bufs = (rows_a, rows_b, rows_c, rows_d)

    n_full = n_tors // _CHUNK
    tail = n_tors - n_full * _CHUNK
    assert tail % _L == 0 and tail % 8 == 0

    def process(base, length):
        ngrp = length // _L
        pltpu.sync_copy(tors_hbm.at[pl.ds(base * 4, length * 4)],
                        tors_v.at[pl.ds(0, length * 4)])

        def deint(g, carry):
            fidx = g * (4 * _L) + iota * 4
            for c in range(4):
                v = plsc.load_gather(tors_v, [fidx + c])
                idx_bufs[c][pl.ds(g * _L, _L)] = v
            return carry

        lax.fori_loop(0, ngrp, deint, 0)

        handles = []
        for ib, rb_ in zip(idx_bufs, row_bufs):
            src = coords_hbm.at[ib.at[pl.ds(0, length)]]
            dst = rb_.at[pl.ds(0, length)]
            handles.append(pltpu.async_copy(src, dst, sem))
        for h in handles:
            h.wait()

        def compute(g, carry):
            rb = g * _L
            ridx = rb + iota
            r = []
            for buf in row_bufs:
                r.append([plsc.load_gather(buf, [ridx, col[c]])
                          for c in range(3)])
            ri, rj, rk, rl = r
            b1 = [rj[c] - ri[c] for c in range(3)]
            b2 = [rk[c] - rj[c] for c in range(3)]
            b3 = [rl[c] - rk[c] for c in range(3)]

            def cross(u, v):
                return [u[1] * v[2] - u[2] * v[1],
                        u[2] * v[0] - u[0] * v[2],
                        u[0] * v[1] - u[1] * v[0]]

            n1 = cross(b1, b2)
            n2 = cross(b2, b3)
            d12 = n1[0] * n2[0] + n1[1] * n2[1] + n1[2] * n2[2]
            q1 = n1[0] * n1[0] + n1[1] * n1[1] + n1[2] * n1[2]
            q2 = n2[0] * n2[0] + n2[1] * n2[1] + n2[2] * n2[2]
            d3 = n1[0] * b3[0] + n1[1] * b3[1] + n1[2] * b3[2]
            cosv = d12 * _rsqrt(q1 * q2)
            cosv = jnp.clip(cosv, -1.0, 1.0)
            phi = _acos(cosv)
            phi = jnp.where(d3 > 0.0, phi, -phi)
            out_v[pl.ds(rb, _L)] = phi
            return carry

        lax.fori_loop(0, ngrp, compute, 0)
        pltpu.sync_copy(out_v.at[pl.ds(0, length)],
                        out_hbm.at[pl.ds(base, length)])

    max_t = (n_full + _NW - 1) // _NW

    def chunk_loop(t, carry):
        cid = w + _NW * t

        @pl.when(cid < n_full)
        def _():
            process(pl.multiple_of(cid * _CHUNK, _CHUNK), _CHUNK)

        return carry

    lax.fori_loop(0, max_t, chunk_loop, 0)

    if tail:
        @pl.when(w == _NW - 1)
        def _():
            process(n_full * _CHUNK, tail)


@jax.jit
def _torsion_sc(coords, torsions):
    n_tors = torsions.shape[0]
    tors_flat = torsions.reshape(-1)
    mesh = plsc.VectorSubcoreMesh(core_axis_name="c", subcore_axis_name="s")
    kern = pl.kernel(
        functools.partial(_body, n_tors=n_tors),
        out_type=jax.ShapeDtypeStruct((n_tors,), jnp.float32),
        mesh=mesh,
        scratch_types=[
            pltpu.VMEM((_CHUNK * 4,), jnp.int32),
            pltpu.VMEM((_CHUNK,), jnp.int32),
            pltpu.VMEM((_CHUNK,), jnp.int32),
            pltpu.VMEM((_CHUNK,), jnp.int32),
            pltpu.VMEM((_CHUNK,), jnp.int32),
            pltpu.VMEM((_CHUNK, 3), jnp.float32),
            pltpu.VMEM((_CHUNK, 3), jnp.float32),
            pltpu.VMEM((_CHUNK, 3), jnp.float32),
            pltpu.VMEM((_CHUNK, 3), jnp.float32),
            pltpu.VMEM((_CHUNK,), jnp.float32),
            pltpu.SemaphoreType.DMA,
        ],
        compiler_params=pltpu.CompilerParams(
            needs_layout_passes=False, use_tc_tiling_on_sc=False),
    )
    return kern(coords, tors_flat)


def kernel(coords, torsions):
    return _torsion_sc(coords, torsions)
